# baseline (speedup 1.0000x reference)
import jax
import jax.numpy as jnp
import numpy as np
from jax.experimental import pallas as pl
from jax.experimental.pallas import tpu as pltpu

_B = 128
_V = 100000
_BK = 2048
_NB = (_V + _BK - 1) // _BK

_K1 = np.uint32(0)
_K2 = np.uint32(42)
_K3 = np.uint32(_K1 ^ _K2 ^ np.uint32(0x1BD11BDA))
_TINY = np.float32(np.finfo(np.float32).tiny)
_SCALE = np.float32(np.float32(1.0) - _TINY)
_R1 = (13, 15, 26, 6)
_R2 = (17, 29, 16, 24)


def _threefry_round(x0, x1, r):
    x0 = x0 + x1
    x1 = (x1 << np.uint32(r)) | (x1 >> np.uint32(32 - r))
    return x0, x0 ^ x1


def _gumbel_bits(lo):
    x0 = jnp.zeros_like(lo) + _K1
    x1 = lo + _K2
    for r in _R1:
        x0, x1 = _threefry_round(x0, x1, r)
    x0 = x0 + _K2
    x1 = x1 + _K3 + np.uint32(1)
    for r in _R2:
        x0, x1 = _threefry_round(x0, x1, r)
    x0 = x0 + _K3
    x1 = x1 + _K1 + np.uint32(2)
    for r in _R1:
        x0, x1 = _threefry_round(x0, x1, r)
    x0 = x0 + _K1
    x1 = x1 + _K2 + np.uint32(3)
    for r in _R2:
        x0, x1 = _threefry_round(x0, x1, r)
    x0 = x0 + _K2
    x1 = x1 + _K3 + np.uint32(4)
    for r in _R1:
        x0, x1 = _threefry_round(x0, x1, r)
    x0 = x0 + _K3
    x1 = x1 + _K1 + np.uint32(5)
    return x0 ^ x1


def _sample_kernel(logits_ref, out_ref, max_ref, idx_ref):
    j = pl.program_id(0)

    @pl.when(j == 0)
    def _init():
        max_ref[...] = jnp.full((_B, 1), -jnp.inf, jnp.float32)
        idx_ref[...] = jnp.zeros((_B, 1), jnp.int32)

    row = jax.lax.broadcasted_iota(jnp.uint32, (_B, _BK), 0)
    col = jax.lax.broadcasted_iota(jnp.int32, (_B, _BK), 1) + j * _BK
    lo = row * np.uint32(_V) + col.astype(jnp.uint32)

    bits = _gumbel_bits(lo)
    fb = (bits >> np.uint32(9)) | np.uint32(0x3F800000)
    f = jax.lax.bitcast_convert_type(fb, jnp.float32) - np.float32(1.0)
    u = jnp.maximum(_TINY, f * _SCALE + _TINY)
    g = -jnp.log(-jnp.log(u))

    v = g + logits_ref[...]
    v = jnp.where(col < _V, v, -jnp.inf)

    bmax = jnp.max(v, axis=1, keepdims=True)
    bidx = jnp.min(
        jnp.where(v == bmax, col, jnp.int32(np.iinfo(np.int32).max)),
        axis=1, keepdims=True)

    better = bmax > max_ref[...]
    max_ref[...] = jnp.where(better, bmax, max_ref[...])
    idx_ref[...] = jnp.where(better, bidx, idx_ref[...])

    @pl.when(j == _NB - 1)
    def _fin():
        out_ref[...] = idx_ref[...]


def kernel(logits):
    out = pl.pallas_call(
        _sample_kernel,
        grid=(_NB,),
        in_specs=[pl.BlockSpec((_B, _BK), lambda j: (0, j))],
        out_specs=pl.BlockSpec((_B, 1), lambda j: (0, 0)),
        out_shape=jax.ShapeDtypeStruct((_B, 1), jnp.int32),
        scratch_shapes=[
            pltpu.VMEM((_B, 1), jnp.float32),
            pltpu.VMEM((_B, 1), jnp.int32),
        ],
        compiler_params=pltpu.CompilerParams(
            dimension_semantics=("arbitrary",)),
    )(logits)
    return out.reshape(_B)


# speedup vs baseline: 1.0887x; 1.0887x over previous
import jax
import jax.numpy as jnp
import numpy as np
from jax.experimental import pallas as pl
from jax.experimental.pallas import tpu as pltpu

_B = 128
_V = 100000
_BK = 2048
_NB = (_V + _BK - 1) // _BK

_K2 = np.uint32(42)
_K3 = np.uint32(42 ^ 0x1BD11BDA)
_TINY = np.float32(np.finfo(np.float32).tiny)
_R1 = (13, 15, 26, 6)
_R2 = (17, 29, 16, 24)
_IMAX = np.int32(np.iinfo(np.int32).max)


def _rotl(x, r):
    return (x << np.uint32(r)) | (x >> np.uint32(32 - r))


def _round(x0, x1, r):
    x0 = x0 + x1
    x1 = _rotl(x1, r)
    return x0, x0 ^ x1


def _gumbel_bits(x1):
    x0 = x1
    x1 = _rotl(x1, _R1[0]) ^ x0
    for r in _R1[1:]:
        x0, x1 = _round(x0, x1, r)
    x0 = x0 + _K2
    x1 = x1 + np.uint32(_K3 + np.uint32(1))
    for r in _R2:
        x0, x1 = _round(x0, x1, r)
    x0 = x0 + _K3
    x1 = x1 + np.uint32(2)
    for r in _R1:
        x0, x1 = _round(x0, x1, r)
    x1 = x1 + np.uint32(45)
    for r in _R2:
        x0, x1 = _round(x0, x1, r)
    x0 = x0 + _K2
    x1 = x1 + np.uint32(_K3 + np.uint32(4))
    for r in _R1:
        x0, x1 = _round(x0, x1, r)
    x0 = x0 + _K3
    x1 = x1 + np.uint32(5)
    return x0 ^ x1


def _values(logits_ref, cnt_ref, j):
    base = (j * _BK + jnp.int32(42)).astype(jnp.uint32)
    bits = _gumbel_bits(cnt_ref[...] + base)
    fb = (bits >> np.uint32(9)) | np.uint32(0x3F800000)
    f = jax.lax.bitcast_convert_type(fb, jnp.float32) - np.float32(1.0)
    u = jnp.maximum(_TINY, f + _TINY)
    g = -jnp.log(-jnp.log(u))
    return g + logits_ref[...]


def _sample_kernel(logits_ref, out_ref, m_ref, wb_ref, cnt_ref):
    j = pl.program_id(0)

    @pl.when(j == 0)
    def _init():
        row = jax.lax.broadcasted_iota(jnp.uint32, (_B, _BK), 0)
        lane = jax.lax.broadcasted_iota(jnp.uint32, (_B, _BK), 1)
        cnt_ref[...] = row * np.uint32(_V) + lane
        m_ref[...] = _values(logits_ref, cnt_ref, j)
        wb_ref[...] = jnp.zeros((_B, _BK), jnp.int32)

    @pl.when(jnp.logical_and(j > 0, j < _NB - 1))
    def _merge():
        v = _values(logits_ref, cnt_ref, j)
        better = v > m_ref[...]
        m_ref[...] = jnp.maximum(m_ref[...], v)
        wb_ref[...] = jnp.where(better, j, wb_ref[...])

    @pl.when(j == _NB - 1)
    def _last():
        v = _values(logits_ref, cnt_ref, j)
        lane = jax.lax.broadcasted_iota(jnp.int32, (_B, _BK), 1)
        v = jnp.where(lane < _V - (_NB - 1) * _BK, v, -jnp.inf)
        better = v > m_ref[...]
        m = jnp.maximum(m_ref[...], v)
        wb = jnp.where(better, j, wb_ref[...])
        mmax = jnp.max(m, axis=1, keepdims=True)
        col = wb * _BK + lane
        out_ref[...] = jnp.min(
            jnp.where(m == mmax, col, _IMAX), axis=1, keepdims=True)


def kernel(logits):
    out = pl.pallas_call(
        _sample_kernel,
        grid=(_NB,),
        in_specs=[pl.BlockSpec((_B, _BK), lambda j: (0, j))],
        out_specs=pl.BlockSpec((_B, 1), lambda j: (0, 0)),
        out_shape=jax.ShapeDtypeStruct((_B, 1), jnp.int32),
        scratch_shapes=[
            pltpu.VMEM((_B, _BK), jnp.float32),
            pltpu.VMEM((_B, _BK), jnp.int32),
            pltpu.VMEM((_B, _BK), jnp.uint32),
        ],
        compiler_params=pltpu.CompilerParams(
            dimension_semantics=("arbitrary",)),
    )(logits)
    return out.reshape(_B)
